# Initial kernel scaffold; baseline (speedup 1.0000x reference)
import jax
import jax.numpy as jnp
from jax.experimental import pallas as pl


def kernel(x, edge_index, W1, b1, W2, b2):
    raise NotImplementedError("write your pallas kernel here")


# speedup vs baseline: 20.8000x; 20.8000x over previous
import functools

import jax
import jax.numpy as jnp
from jax import lax
from jax.experimental import pallas as pl
from jax.experimental.pallas import tpu as pltpu
from jax.experimental.pallas import tpu_sc as plsc

N = 10000
NPAD = 10240
D = 128
NC, NS, L = 2, 16, 16
CHUNK = 128
CH = 80
EPAD = NC * NS * CH * CHUNK
RPT = NPAD // NS
WD = 128


@functools.cache
def _sc_kernels():
    mesh = plsc.VectorSubcoreMesh(
        core_axis_name="c", subcore_axis_name="s", num_cores=NC, num_subcores=NS)

    @functools.partial(
        pl.kernel,
        out_type=jax.ShapeDtypeStruct((NC, NPAD, WD), jnp.float32),
        mesh=mesh,
        scratch_types=[
            pltpu.VMEM((CHUNK,), jnp.int32),
            pltpu.VMEM((CHUNK, WD), jnp.float32),
            pltpu.VMEM_SHARED((NPAD, WD), jnp.float32),
        ],
    )
    def deg_kernel(dst_hbm, zeros_hbm, ones_hbm, out_hbm, dst_v, ones_v, hist_sh):
        cid = lax.axis_index("c")
        sid = lax.axis_index("s")
        sl = pl.ds(sid * RPT, RPT)
        pltpu.sync_copy(ones_hbm, ones_v)

        def zbody(i, carry):
            pltpu.sync_copy(ones_v, hist_sh.at[pl.ds(sid * RPT + i * CHUNK, CHUNK), :])
            return carry

        pltpu.sync_copy(zeros_hbm, ones_v)
        lax.fori_loop(0, RPT // CHUNK, zbody, 0)
        pltpu.sync_copy(ones_hbm, ones_v)
        plsc.subcore_barrier()

        def body(j, carry):
            pltpu.sync_copy(dst_hbm.at[cid, sid, j], dst_v)
            pltpu.sync_copy(ones_v, hist_sh.at[dst_v], add=True)
            return carry

        lax.fori_loop(0, CH, body, 0)
        plsc.subcore_barrier()
        pltpu.sync_copy(hist_sh.at[sl, :], out_hbm.at[cid, sl, :])

    @functools.partial(
        pl.kernel,
        out_type=jax.ShapeDtypeStruct((NC, NPAD, D), jnp.float32),
        mesh=mesh,
        scratch_types=[
            pltpu.VMEM((CHUNK,), jnp.int32),
            pltpu.VMEM((CHUNK,), jnp.int32),
            pltpu.VMEM((CHUNK,), jnp.int32),
            pltpu.VMEM((CHUNK,), jnp.int32),
            pltpu.VMEM((CHUNK, D), jnp.float32),
            pltpu.VMEM((CHUNK, D), jnp.float32),
            pltpu.VMEM_SHARED((NPAD, D), jnp.float32),
            pltpu.SemaphoreType.DMA,
            pltpu.SemaphoreType.DMA,
            pltpu.SemaphoreType.DMA,
            pltpu.SemaphoreType.DMA,
        ],
    )
    def agg_kernel(y_hbm, src_hbm, dst_hbm, zeros_hbm, out_hbm,
                   src_a, dst_a, src_b, dst_b, rows_a, rows_b, agg_sh,
                   gsem_a, gsem_b, ssem_a, ssem_b):
        cid = lax.axis_index("c")
        sid = lax.axis_index("s")
        sl = pl.ds(sid * RPT, RPT)

        pltpu.sync_copy(zeros_hbm, rows_a)

        def zbody(i, carry):
            pltpu.sync_copy(rows_a, agg_sh.at[pl.ds(sid * RPT + i * CHUNK, CHUNK), :])
            return carry

        lax.fori_loop(0, RPT // CHUNK, zbody, 0)
        plsc.subcore_barrier()

        def stage_fire(j, src_v, dst_v, rows_v, gsem):
            pltpu.sync_copy(src_hbm.at[cid, sid, j], src_v)
            pltpu.sync_copy(dst_hbm.at[cid, sid, j], dst_v)
            pltpu.async_copy(y_hbm.at[src_v], rows_v, gsem)

        stage_fire(0, src_a, dst_a, rows_a, gsem_a)

        def body(g, carry):
            j = 2 * g
            stage_fire(j + 1, src_b, dst_b, rows_b, gsem_b)
            pltpu.make_async_copy(y_hbm.at[src_a], rows_a, gsem_a).wait()
            pltpu.async_copy(rows_a, agg_sh.at[dst_a], ssem_a, add=True)

            @pl.when(j + 2 < CH)
            def _():
                pltpu.make_async_copy(rows_a, agg_sh.at[dst_a], ssem_a).wait()
                stage_fire(j + 2, src_a, dst_a, rows_a, gsem_a)

            @pl.when(j + 2 >= CH)
            def _():
                pltpu.make_async_copy(rows_a, agg_sh.at[dst_a], ssem_a).wait()

            pltpu.make_async_copy(y_hbm.at[src_b], rows_b, gsem_b).wait()
            pltpu.async_copy(rows_b, agg_sh.at[dst_b], ssem_b, add=True)
            pltpu.make_async_copy(rows_b, agg_sh.at[dst_b], ssem_b).wait()
            return carry

        lax.fori_loop(0, CH // 2, body, 0)
        plsc.subcore_barrier()
        pltpu.sync_copy(agg_sh.at[sl, :], out_hbm.at[cid, sl, :])

    return deg_kernel, agg_kernel


def _sc_deg(dst, zeros, ones):
    return _sc_kernels()[0](dst, zeros, ones)


def _sc_agg(y, src, dst, zeros):
    return _sc_kernels()[1](y, src, dst, zeros)


BR = 1024
GRID = NPAD // BR


def _tc1_body(degp_ref, x_ref, w_ref, y_ref, dis_ref):
    d = degp_ref[0, :, 0:1] + degp_ref[1, :, 0:1] + 1.0
    dis = lax.rsqrt(d)
    xw = jnp.dot(x_ref[...], w_ref[...], preferred_element_type=jnp.float32, precision=lax.Precision.HIGHEST)
    y_ref[...] = xw * dis
    dis_ref[...] = dis


def _tc2_body(aggp_ref, y1_ref, dis_ref, b1_ref, w2_ref, y2_ref):
    agg = aggp_ref[0] + aggp_ref[1] + y1_ref[...]
    hidden = agg * dis_ref[...] + b1_ref[...]
    h = jnp.maximum(hidden, 0.0)
    y2_ref[...] = jnp.dot(h, w2_ref[...], preferred_element_type=jnp.float32, precision=lax.Precision.HIGHEST) * dis_ref[...]


def _tc3_body(aggp_ref, y2_ref, dis_ref, b2_ref, logp_ref, out_ref):
    out = (aggp_ref[0] + aggp_ref[1] + y2_ref[...]) * dis_ref[...] + b2_ref[...]
    m = jnp.max(out, axis=1, keepdims=True)
    lse = jnp.log(jnp.sum(jnp.exp(out - m), axis=1, keepdims=True)) + m
    logp_ref[...] = out - lse
    out_ref[...] = out


_row_spec = pl.BlockSpec((BR, D), lambda i: (i, 0))
_dis_spec = pl.BlockSpec((BR, 1), lambda i: (i, 0))
_w_spec = pl.BlockSpec((D, D), lambda i: (0, 0))
_b_spec = pl.BlockSpec((1, D), lambda i: (0, 0))
_aggp_spec = pl.BlockSpec((NC, BR, D), lambda i: (0, i, 0))

_tc1 = pl.pallas_call(
    _tc1_body,
    grid=(GRID,),
    in_specs=[pl.BlockSpec((NC, BR, WD), lambda i: (0, i, 0)), _row_spec, _w_spec],
    out_specs=[_row_spec, _dis_spec],
    out_shape=[jax.ShapeDtypeStruct((NPAD, D), jnp.float32),
               jax.ShapeDtypeStruct((NPAD, 1), jnp.float32)],
)

_tc2 = pl.pallas_call(
    _tc2_body,
    grid=(GRID,),
    in_specs=[_aggp_spec, _row_spec, _dis_spec, _b_spec, _w_spec],
    out_specs=_row_spec,
    out_shape=jax.ShapeDtypeStruct((NPAD, D), jnp.float32),
)

_tc3 = pl.pallas_call(
    _tc3_body,
    grid=(GRID,),
    in_specs=[_aggp_spec, _row_spec, _dis_spec, _b_spec],
    out_specs=[_row_spec, _row_spec],
    out_shape=[jax.ShapeDtypeStruct((NPAD, D), jnp.float32),
               jax.ShapeDtypeStruct((NPAD, D), jnp.float32)],
)


def kernel(x, edge_index, W1, b1, W2, b2):
    ei = edge_index.astype(jnp.int32)
    n_edges = ei.shape[1]
    pad = EPAD - n_edges
    pad_src = jnp.arange(pad, dtype=jnp.int32) % N
    pad_dst = N + jnp.arange(pad, dtype=jnp.int32) % (NPAD - N)
    src = jnp.concatenate([ei[0], pad_src]).reshape(NC, NS, CH, CHUNK)
    dst = jnp.concatenate([ei[1], pad_dst]).reshape(NC, NS, CH, CHUNK)

    x_pad = jnp.pad(x, ((0, NPAD - N), (0, 0)))
    zeros = jnp.zeros((CHUNK, D), jnp.float32)
    ones = jnp.ones((CHUNK, WD), jnp.float32)
    zeros_w = jnp.zeros((CHUNK, WD), jnp.float32)
    b1r = b1.reshape(1, D)
    b2r = b2.reshape(1, D)

    degp = _sc_deg(dst, zeros_w, ones)
    y1, dis = _tc1(degp, x_pad, W1)
    aggp1 = _sc_agg(y1, src, dst, zeros)
    y2 = _tc2(aggp1, y1, dis, b1r, W2)
    aggp2 = _sc_agg(y2, src, dst, zeros)
    logp, out = _tc3(aggp2, y2, dis, b2r)
    return (logp[:N], out[:N])
